# baseline (speedup 1.0000x reference)
import jax
import jax.numpy as jnp
from jax.experimental import pallas as pl
from jax.experimental.pallas import tpu as pltpu

N = 32768
D = 768
E = 16

_R = 128
_C = N // _R
_FB = 4096
_NB = N // _FB


def _prep_kernel(inds_ref, out_ref):
    inds = inds_ref[...]
    lin = (jax.lax.broadcasted_iota(jnp.int32, (_R, _C), 0) * _C
           + jax.lax.broadcasted_iota(jnp.int32, (_R, _C), 1))
    counts = []
    lasts = []
    for j in range(E):
        m = inds == j
        counts.append(jnp.sum(m.astype(jnp.int32)))
        lasts.append(jnp.max(jnp.where(m, lin, -1)))
    cums = []
    acc = counts[0]
    cums.append(acc)
    for j in range(1, E):
        acc = acc + counts[j]
        cums.append(acc)
    e_sel = []
    valid = []
    for j in range(E):
        e = counts[0] * 0
        for k in range(E):
            e = e + (cums[k] <= lasts[j]).astype(jnp.int32)
        e_sel.append(jnp.minimum(e, E - 1))
        valid.append((counts[j] > 0).astype(jnp.int32))
    used = []
    for e in range(E):
        u = counts[0] * 0
        for j in range(E):
            u = u | (valid[j] & (e_sel[j] == e).astype(jnp.int32))
        used.append(u)
    rank = []
    r = counts[0] * 0
    for e in range(E):
        rank.append(r)
        r = r + used[e]
    num_used = r
    wsel = []
    active = []
    for t in range(E):
        idx = jnp.minimum(jnp.int32(t), num_used - 1)
        w = counts[0] * 0
        for e in range(E):
            w = w + e * used[e] * (rank[e] == idx).astype(jnp.int32)
        wsel.append(w)
        active.append((t < num_used).astype(jnp.int32))
    out_ref[...] = jnp.zeros((8, 128), jnp.int32)
    out_ref[0:1, 0:E] = jnp.stack(e_sel).reshape(1, E)
    out_ref[1:2, 0:E] = jnp.stack(valid).reshape(1, E)
    out_ref[2:3, 0:E] = jnp.stack(wsel).reshape(1, E)
    out_ref[3:4, 0:E] = jnp.stack(active).reshape(1, E)


def _main_kernel(esel_ref, valid_ref, wsel_ref,
                 x_ref, wa_ref, wb_ref, b_ref, out_ref, rows_ref):
    t = pl.program_id(0)

    @pl.when(t == 0)
    def _():
        rows_ref[...] = jnp.zeros_like(rows_ref)

    def slot(w_ref, slot_idx):
        cur = wsel_ref[slot_idx]
        y = jax.lax.dot_general(
            x_ref[...], w_ref[0], (((1,), (1,)), ((), ())),
            preferred_element_type=jnp.float32)
        onehot = (jax.lax.broadcasted_iota(jnp.int32, (1, E), 1)
                  == cur).astype(jnp.float32)
        y = y + jax.lax.dot_general(
            onehot, b_ref[...], (((1,), (0,)), ((), ())),
            preferred_element_type=jnp.float32)
        for j in range(E):
            @pl.when((valid_ref[j] == 1) & (esel_ref[j] == cur))
            def _():
                rows_ref[j:j + 1, :] = y[j:j + 1, :]

    slot(wa_ref, 2 * t)
    slot(wb_ref, 2 * t + 1)

    out_ref[...] = jnp.zeros_like(out_ref)

    @pl.when(t == _NB - 1)
    def _():
        out_ref[0:E, :] = rows_ref[...]


def kernel(features, inds, W, b):
    inds2d = inds.astype(jnp.int32).reshape(_R, _C)

    prep = pl.pallas_call(
        _prep_kernel,
        out_shape=jax.ShapeDtypeStruct((8, 128), jnp.int32),
    )(inds2d)
    e_sel = prep[0, :E]
    valid = prep[1, :E]
    wsel = prep[2, :E]

    out = pl.pallas_call(
        _main_kernel,
        grid_spec=pltpu.PrefetchScalarGridSpec(
            num_scalar_prefetch=3,
            grid=(_NB,),
            in_specs=[
                pl.BlockSpec((E, D), lambda t, es, va, ws: (0, 0)),
                pl.BlockSpec((1, D, D),
                             lambda t, es, va, ws: (ws[2 * t], 0, 0)),
                pl.BlockSpec((1, D, D),
                             lambda t, es, va, ws: (ws[2 * t + 1], 0, 0)),
                pl.BlockSpec((E, D), lambda t, es, va, ws: (0, 0)),
            ],
            out_specs=pl.BlockSpec(
                (_FB, D), lambda t, es, va, ws: ((t + 1) % _NB, 0)),
            scratch_shapes=[pltpu.VMEM((E, D), jnp.float32)],
        ),
        out_shape=jax.ShapeDtypeStruct((N, D), jnp.float32),
    )(e_sel, valid, wsel, features, W, W, b)
    return out


# speedup vs baseline: 1.0323x; 1.0323x over previous
import jax
import jax.numpy as jnp
from jax.experimental import pallas as pl
from jax.experimental.pallas import tpu as pltpu

N = 32768
D = 768
E = 16

_R = 128
_C = N // _R
_FB = 4096
_NB = N // _FB


def _prep_kernel(inds_ref, out_ref):
    inds = inds_ref[...]
    lin = (jax.lax.broadcasted_iota(jnp.int32, (_R, _C), 0) * _C
           + jax.lax.broadcasted_iota(jnp.int32, (_R, _C), 1))
    counts = []
    lasts = []
    for j in range(E):
        m = inds == j
        counts.append(jnp.sum(m.astype(jnp.int32)))
        lasts.append(jnp.max(jnp.where(m, lin, -1)))
    cums = []
    acc = counts[0]
    cums.append(acc)
    for j in range(1, E):
        acc = acc + counts[j]
        cums.append(acc)
    e_sel = []
    valid = []
    for j in range(E):
        e = counts[0] * 0
        for k in range(E):
            e = e + (cums[k] <= lasts[j]).astype(jnp.int32)
        e_sel.append(jnp.minimum(e, E - 1))
        valid.append((counts[j] > 0).astype(jnp.int32))
    used = []
    for e in range(E):
        u = counts[0] * 0
        for j in range(E):
            u = u | (valid[j] & (e_sel[j] == e).astype(jnp.int32))
        used.append(u)
    rank = []
    r = counts[0] * 0
    for e in range(E):
        rank.append(r)
        r = r + used[e]
    num_used = r
    wsel = []
    active = []
    for t in range(E):
        idx = jnp.minimum(jnp.int32(t), num_used - 1)
        w = counts[0] * 0
        for e in range(E):
            w = w + e * used[e] * (rank[e] == idx).astype(jnp.int32)
        wsel.append(w)
        active.append((t < num_used).astype(jnp.int32))
    out_ref[...] = jnp.zeros((8, 128), jnp.int32)
    out_ref[0:1, 0:E] = jnp.stack(e_sel).reshape(1, E)
    out_ref[1:2, 0:E] = jnp.stack(valid).reshape(1, E)
    out_ref[2:3, 0:E] = jnp.stack(wsel).reshape(1, E)
    out_ref[3:4, 0:E] = jnp.stack(active).reshape(1, E)


def _main_kernel(meta_ref, x_ref, wa_ref, wb_ref, b_ref, out_ref, rows_ref):
    t = pl.program_id(0)

    @pl.when(t == 0)
    def _():
        rows_ref[...] = jnp.zeros_like(rows_ref)

    def slot(w_ref, slot_idx):
        cur = meta_ref[2, slot_idx]
        y = jax.lax.dot_general(
            x_ref[...], w_ref[0], (((1,), (1,)), ((), ())),
            preferred_element_type=jnp.float32)
        onehot = (jax.lax.broadcasted_iota(jnp.int32, (1, E), 1)
                  == cur).astype(jnp.float32)
        y = y + jax.lax.dot_general(
            onehot, b_ref[...], (((1,), (0,)), ((), ())),
            preferred_element_type=jnp.float32)
        for j in range(E):
            @pl.when((meta_ref[1, j] == 1) & (meta_ref[0, j] == cur))
            def _():
                rows_ref[j:j + 1, :] = y[j:j + 1, :]

    slot(wa_ref, 2 * t)
    slot(wb_ref, 2 * t + 1)

    out_ref[...] = jnp.zeros_like(out_ref)

    @pl.when(t == _NB - 1)
    def _():
        out_ref[0:E, :] = rows_ref[...]


def kernel(features, inds, W, b):
    inds2d = inds.astype(jnp.int32).reshape(_R, _C)

    prep = pl.pallas_call(
        _prep_kernel,
        out_shape=jax.ShapeDtypeStruct((8, 128), jnp.int32),
    )(inds2d)
    out = pl.pallas_call(
        _main_kernel,
        grid_spec=pltpu.PrefetchScalarGridSpec(
            num_scalar_prefetch=1,
            grid=(_NB,),
            in_specs=[
                pl.BlockSpec((E, D), lambda t, m: (0, 0)),
                pl.BlockSpec((1, D, D), lambda t, m: (m[2, 2 * t], 0, 0)),
                pl.BlockSpec((1, D, D), lambda t, m: (m[2, 2 * t + 1], 0, 0)),
                pl.BlockSpec((E, D), lambda t, m: (0, 0)),
            ],
            out_specs=pl.BlockSpec(
                (_FB, D), lambda t, m: ((t + 1) % _NB, 0)),
            scratch_shapes=[pltpu.VMEM((E, D), jnp.float32)],
        ),
        out_shape=jax.ShapeDtypeStruct((N, D), jnp.float32),
    )(prep, features, W, W, b)
    return out
